# baseline (speedup 1.0000x reference)
import functools

import jax
import jax.numpy as jnp
from jax import lax
from jax.experimental import pallas as pl
from jax.experimental.pallas import tpu as pltpu
from jax.experimental.pallas import tpu_sc as plsc

K = 8192
D = 256
N = 16384
HW = 1024
BN = 1024
BK = 2048
NKB = K // BK
W_BOUNDS = (0, 2736, 5472, K)
NWIN = 3


def _argmin_body(x_ref, sx2_ref, se2_ref, e_ref, out_ref, dscr, *wscr):
    wmin = wscr[:NWIN]
    widx = wscr[NWIN:]
    kk = pl.program_id(1)

    @pl.when(kk == 0)
    def _init():
        for w in range(NWIN):
            wmin[w][...] = jnp.full((1, BN), jnp.inf, dtype=jnp.float32)
            widx[w][...] = jnp.zeros((1, BN), dtype=jnp.int32)

    @pl.when(kk > 0)
    def _process():
        kp = kk - 1
        lo = kp * BK
        d = dscr[...]
        srow = lax.broadcasted_iota(jnp.int32, (BK, BN), 0).astype(jnp.float32)
        bigf = jnp.float32(K)
        for w in range(NWIN):
            full = (lo >= W_BOUNDS[w]) & (lo + BK <= W_BOUNDS[w + 1])
            overlap = (lo < W_BOUNDS[w + 1]) & (lo + BK > W_BOUNDS[w])
            part = overlap & jnp.logical_not(full)

            @pl.when(full)
            def _(w=w, d=d, srow=srow):
                dmin = jnp.min(d, axis=0, keepdims=True)
                li = jnp.min(jnp.where(d == dmin, srow, bigf),
                             axis=0, keepdims=True)
                lt = dmin < wmin[w][...]
                widx[w][...] = jnp.where(
                    lt, li.astype(jnp.int32) + lo, widx[w][...])
                wmin[w][...] = jnp.where(lt, dmin, wmin[w][...])

            @pl.when(part)
            def _(w=w, d=d, srow=srow):
                wlo = W_BOUNDS[w] - lo
                whi = W_BOUNDS[w + 1] - lo
                inw = ((srow >= wlo.astype(jnp.float32))
                       & (srow < whi.astype(jnp.float32)))
                dw = jnp.where(inw, d, jnp.inf)
                dmin = jnp.min(dw, axis=0, keepdims=True)
                li = jnp.min(jnp.where(dw == dmin, srow, bigf),
                             axis=0, keepdims=True)
                lt = dmin < wmin[w][...]
                widx[w][...] = jnp.where(
                    lt, li.astype(jnp.int32) + lo, widx[w][...])
                wmin[w][...] = jnp.where(lt, dmin, wmin[w][...])

    @pl.when(kk < NKB)
    def _compute():
        mm = lax.dot_general(
            e_ref[...], x_ref[0],
            dimension_numbers=(((1,), (0,)), ((), ())),
            preferred_element_type=jnp.float32)
        dscr[...] = (sx2_ref[0] + se2_ref[0]) - 2.0 * mm

    @pl.when(kk == NKB)
    def _flush():
        accv = jnp.full((1, BN), jnp.inf, dtype=jnp.float32)
        acci = jnp.zeros((1, BN), dtype=jnp.int32)
        for w in range(NWIN):
            mv = wmin[w][...]
            iv = widx[w][...]
            lt = mv < accv
            eq = (mv == accv) & (iv < acci)
            acci = jnp.where(lt | eq, iv, acci)
            accv = jnp.where(lt, mv, accv)
            accv = accv.astype(jnp.bfloat16).astype(jnp.float32)
        out_ref[0] = acci


def _argmin_call(x_raw, sx2, se2, emb):
    kmap = lambda n, k: (jnp.minimum(k, NKB - 1), 0)
    return pl.pallas_call(
        _argmin_body,
        grid=(N // BN, NKB + 1),
        in_specs=[
            pl.BlockSpec((1, D, BN), lambda n, k: (n, 0, 0)),
            pl.BlockSpec((1, 1, BN), lambda n, k: (n, 0, 0)),
            pl.BlockSpec((1, BK, 1), lambda n, k: (jnp.minimum(k, NKB - 1), 0, 0)),
            pl.BlockSpec((BK, D), kmap),
        ],
        out_specs=pl.BlockSpec((1, 1, BN), lambda n, k: (n, 0, 0)),
        out_shape=jax.ShapeDtypeStruct((N // BN, 1, BN), jnp.int32),
        scratch_shapes=[pltpu.VMEM((BK, BN), jnp.float32)]
                      + [pltpu.VMEM((1, BN), jnp.float32) for _ in range(NWIN)]
                      + [pltpu.VMEM((1, BN), jnp.int32) for _ in range(NWIN)],
        compiler_params=pltpu.CompilerParams(
            dimension_semantics=("parallel", "arbitrary")),
    )(x_raw, sx2, se2, emb)


def _make_gather():
    info = plsc.get_sparse_core_info()
    nc, ns = info.num_cores, info.num_subcores
    nw = nc * ns
    rows_per_w = N // nw
    chunk = 128
    nchunk = rows_per_w // chunk
    idx_rows = N // chunk
    mesh = plsc.VectorSubcoreMesh(core_axis_name="c", subcore_axis_name="s")

    @functools.partial(
        pl.kernel,
        out_type=jax.ShapeDtypeStruct((N, D), jnp.float32),
        mesh=mesh,
        scratch_types=[
            pltpu.VMEM((nchunk, chunk), jnp.int32),
            pltpu.VMEM((chunk, D), jnp.float32),
            pltpu.SemaphoreType.DMA,
        ],
    )
    def gather(table_hbm, idx_hbm, out_hbm, idx_v, rows_v, sem):
        wid = lax.axis_index("s") * nc + lax.axis_index("c")
        base = wid * rows_per_w
        pltpu.sync_copy(idx_hbm.at[pl.ds(wid * nchunk, nchunk)], idx_v)
        for j in range(nchunk):
            pltpu.async_copy(table_hbm.at[idx_v.at[j]], rows_v, sem).wait()
            pltpu.sync_copy(rows_v, out_hbm.at[pl.ds(base + j * chunk, chunk)])

    def run(emb, idx_flat):
        return gather(emb, idx_flat.reshape(idx_rows, chunk))

    return run


_gather_cache = []


def _gather_call(emb, idx_flat):
    if not _gather_cache:
        _gather_cache.append(_make_gather())
    return _gather_cache[0](emb, idx_flat)


def _loss_body(q_ref, x_ref, qout_ref, loss_ref):
    q = q_ref[...]
    xr = x_ref[0]
    xt = jnp.transpose(xr)
    t = q - xt
    t2 = t * t
    loss_ref[...] = t2 + 0.25 * t2
    qout_ref[0] = xr + jnp.transpose(t)


def _loss_call(q, x_raw):
    return pl.pallas_call(
        _loss_body,
        grid=(N // HW,),
        in_specs=[
            pl.BlockSpec((HW, D), lambda n: (n, 0)),
            pl.BlockSpec((1, D, HW), lambda n: (n, 0, 0)),
        ],
        out_specs=[
            pl.BlockSpec((1, D, HW), lambda n: (n, 0, 0)),
            pl.BlockSpec((HW, D), lambda n: (n, 0)),
        ],
        out_shape=[
            jax.ShapeDtypeStruct((N // HW, D, HW), jnp.float32),
            jax.ShapeDtypeStruct((N, D), jnp.float32),
        ],
        compiler_params=pltpu.CompilerParams(
            dimension_semantics=("parallel",)),
    )(q, x_raw)


def kernel(inputs, embedding_weight):
    b, c, h, w = inputs.shape
    x_raw = inputs.reshape(b, c, h * w)
    sx2 = jnp.sum(jnp.transpose(inputs, (0, 2, 3, 1)) ** 2, axis=3)
    sx2 = sx2.reshape(b, 1, h * w)
    se2 = jnp.sum(embedding_weight ** 2, axis=1)

    idx = _argmin_call(x_raw, sx2, se2.reshape(NKB, BK, 1), embedding_weight)
    idx_flat = idx.reshape(-1)

    q = _gather_call(embedding_weight, idx_flat)
    qout, loss = _loss_call(q, x_raw)

    quantized_out = qout.reshape(b, c, h, w)
    loss_out = loss.reshape(b, h, w, c)
    indices_out = idx_flat.reshape(b, h, w)
    return quantized_out, loss_out, indices_out


# speedup vs baseline: 1.1953x; 1.1953x over previous
import functools

import jax
import jax.numpy as jnp
from jax import lax
from jax.experimental import pallas as pl
from jax.experimental.pallas import tpu as pltpu
from jax.experimental.pallas import tpu_sc as plsc

K = 8192
D = 256
N = 16384
HW = 1024
BN = 1024
BK = 2048
NKB = K // BK
W_BOUNDS = (0, 2736, 5472, K)
NWIN = 3


def _argmin_body(x_ref, sx2_ref, se2_ref, e_ref, out_ref, dscr, *wscr):
    wmin = wscr[:NWIN]
    widx = wscr[NWIN:]
    kk = pl.program_id(1)

    @pl.when(kk == 0)
    def _init():
        for w in range(NWIN):
            wmin[w][...] = jnp.full((1, BN), jnp.inf, dtype=jnp.float32)
            widx[w][...] = jnp.zeros((1, BN), dtype=jnp.int32)

    @pl.when(kk > 0)
    def _process():
        kp = kk - 1
        d = dscr[...]
        bigf = jnp.float32(K)
        for j in range(NKB):
            blo, bhi = j * BK, (j + 1) * BK
            segs = []
            for w in range(NWIN):
                slo = max(blo, W_BOUNDS[w]) - blo
                shi = min(bhi, W_BOUNDS[w + 1]) - blo
                if slo < shi:
                    segs.append((slo, shi, w))

            @pl.when(kp == j)
            def _(j=j, segs=segs, d=d):
                for (slo, shi, w) in segs:
                    dseg = d[slo:shi]
                    srow = lax.broadcasted_iota(
                        jnp.int32, (shi - slo, BN), 0).astype(jnp.float32)
                    dmin = jnp.min(dseg, axis=0, keepdims=True)
                    li = jnp.min(jnp.where(dseg == dmin, srow, bigf),
                                 axis=0, keepdims=True)
                    lt = dmin < wmin[w][...]
                    widx[w][...] = jnp.where(
                        lt, li.astype(jnp.int32) + (j * BK + slo),
                        widx[w][...])
                    wmin[w][...] = jnp.where(lt, dmin, wmin[w][...])

    @pl.when(kk < NKB)
    def _compute():
        mm = lax.dot_general(
            e_ref[...], x_ref[0],
            dimension_numbers=(((1,), (0,)), ((), ())),
            preferred_element_type=jnp.float32)
        dscr[...] = (sx2_ref[0] + se2_ref[0]) - 2.0 * mm

    @pl.when(kk == NKB)
    def _flush():
        accv = jnp.full((1, BN), jnp.inf, dtype=jnp.float32)
        acci = jnp.zeros((1, BN), dtype=jnp.int32)
        for w in range(NWIN):
            mv = wmin[w][...]
            iv = widx[w][...]
            lt = mv < accv
            eq = (mv == accv) & (iv < acci)
            acci = jnp.where(lt | eq, iv, acci)
            accv = jnp.where(lt, mv, accv)
            accv = accv.astype(jnp.bfloat16).astype(jnp.float32)
        out_ref[0] = acci


def _argmin_call(x_raw, sx2, se2, emb):
    kmap = lambda n, k: (jnp.minimum(k, NKB - 1), 0)
    return pl.pallas_call(
        _argmin_body,
        grid=(N // BN, NKB + 1),
        in_specs=[
            pl.BlockSpec((1, D, BN), lambda n, k: (n, 0, 0)),
            pl.BlockSpec((1, 1, BN), lambda n, k: (n, 0, 0)),
            pl.BlockSpec((1, BK, 1), lambda n, k: (jnp.minimum(k, NKB - 1), 0, 0)),
            pl.BlockSpec((BK, D), kmap),
        ],
        out_specs=pl.BlockSpec((1, 1, BN), lambda n, k: (n, 0, 0)),
        out_shape=jax.ShapeDtypeStruct((N // BN, 1, BN), jnp.int32),
        scratch_shapes=[pltpu.VMEM((BK, BN), jnp.float32)]
                      + [pltpu.VMEM((1, BN), jnp.float32) for _ in range(NWIN)]
                      + [pltpu.VMEM((1, BN), jnp.int32) for _ in range(NWIN)],
        compiler_params=pltpu.CompilerParams(
            dimension_semantics=("parallel", "arbitrary")),
    )(x_raw, sx2, se2, emb)


def _make_gather():
    info = plsc.get_sparse_core_info()
    nc, ns = info.num_cores, info.num_subcores
    nw = nc * ns
    rows_per_w = N // nw
    chunk = 128
    nchunk = rows_per_w // chunk
    idx_rows = N // chunk
    mesh = plsc.VectorSubcoreMesh(core_axis_name="c", subcore_axis_name="s")

    @functools.partial(
        pl.kernel,
        out_type=jax.ShapeDtypeStruct((N, D), jnp.float32),
        mesh=mesh,
        scratch_types=[
            pltpu.VMEM((nchunk, chunk), jnp.int32),
            pltpu.VMEM((chunk, D), jnp.float32),
            pltpu.SemaphoreType.DMA,
        ],
    )
    def gather(table_hbm, idx_hbm, out_hbm, idx_v, rows_v, sem):
        wid = lax.axis_index("s") * nc + lax.axis_index("c")
        base = wid * rows_per_w
        pltpu.sync_copy(idx_hbm.at[pl.ds(wid * nchunk, nchunk)], idx_v)
        for j in range(nchunk):
            pltpu.async_copy(table_hbm.at[idx_v.at[j]], rows_v, sem).wait()
            pltpu.sync_copy(rows_v, out_hbm.at[pl.ds(base + j * chunk, chunk)])

    def run(emb, idx_flat):
        return gather(emb, idx_flat.reshape(idx_rows, chunk))

    return run


_gather_cache = []


def _gather_call(emb, idx_flat):
    if not _gather_cache:
        _gather_cache.append(_make_gather())
    return _gather_cache[0](emb, idx_flat)


def _loss_body(q_ref, x_ref, qout_ref, loss_ref):
    q = q_ref[...]
    xr = x_ref[0]
    xt = jnp.transpose(xr)
    t = q - xt
    t2 = t * t
    loss_ref[...] = t2 + 0.25 * t2
    qout_ref[0] = xr + jnp.transpose(t)


def _loss_call(q, x_raw):
    return pl.pallas_call(
        _loss_body,
        grid=(N // HW,),
        in_specs=[
            pl.BlockSpec((HW, D), lambda n: (n, 0)),
            pl.BlockSpec((1, D, HW), lambda n: (n, 0, 0)),
        ],
        out_specs=[
            pl.BlockSpec((1, D, HW), lambda n: (n, 0, 0)),
            pl.BlockSpec((HW, D), lambda n: (n, 0)),
        ],
        out_shape=[
            jax.ShapeDtypeStruct((N // HW, D, HW), jnp.float32),
            jax.ShapeDtypeStruct((N, D), jnp.float32),
        ],
        compiler_params=pltpu.CompilerParams(
            dimension_semantics=("parallel",)),
    )(q, x_raw)


def kernel(inputs, embedding_weight):
    b, c, h, w = inputs.shape
    x_raw = inputs.reshape(b, c, h * w)
    sx2 = jnp.sum(jnp.transpose(inputs, (0, 2, 3, 1)) ** 2, axis=3)
    sx2 = sx2.reshape(b, 1, h * w)
    se2 = jnp.sum(embedding_weight ** 2, axis=1)

    idx = _argmin_call(x_raw, sx2, se2.reshape(NKB, BK, 1), embedding_weight)
    idx_flat = idx.reshape(-1)

    q = _gather_call(embedding_weight, idx_flat)
    qout, loss = _loss_call(q, x_raw)

    quantized_out = qout.reshape(b, c, h, w)
    loss_out = loss.reshape(b, h, w, c)
    indices_out = idx_flat.reshape(b, h, w)
    return quantized_out, loss_out, indices_out
